# baseline (speedup 1.0000x reference)
import jax
import jax.numpy as jnp
from jax.experimental import pallas as pl
from jax.experimental.pallas import tpu as pltpu

_MB = 1024 * 1024
_NSPLIT = 2


def _make_kernel(f_out, nsplit):
    fo = f_out // nsplit

    def _matmul_bias_kernel(x_ref, w_ref, b_ref, o_ref):
        j = pl.program_id(1) if nsplit > 1 else 0
        for jj in range(nsplit):
            @pl.when(j == jj)
            def _(jj=jj):
                w = w_ref[:, jj * fo:(jj + 1) * fo]
                bias = b_ref[:, jj * fo:(jj + 1) * fo].astype(jnp.float32)
                acc = jnp.dot(x_ref[...], w,
                              preferred_element_type=jnp.float32)
                o_ref[...] = (acc + bias).astype(o_ref.dtype)

    return _matmul_bias_kernel


def _pick_tm(n):
    for tm in (2048, 1024, 512, 256, 128, 64, 32, 16, 8):
        if n % tm == 0:
            return tm
    return None


def _linear2d(x2, w, b2):
    n, f_in = x2.shape
    f_out = w.shape[1]
    dtype = x2.dtype
    itemsize = jnp.dtype(dtype).itemsize

    tm = _pick_tm(n)
    if tm is None:
        tm = min(n, 1024)
    nsplit = _NSPLIT if (f_out % (_NSPLIT * 128) == 0) else 1
    fo = f_out // nsplit
    grid = (pl.cdiv(n, tm), nsplit)

    cost = pl.CostEstimate(
        flops=2 * n * f_in * f_out,
        transcendentals=0,
        bytes_accessed=itemsize * (n * f_in + f_in * f_out + f_out + n * f_out),
    )

    return pl.pallas_call(
        _make_kernel(f_out, nsplit),
        out_shape=jax.ShapeDtypeStruct((n, f_out), dtype),
        grid=grid,
        in_specs=[
            pl.BlockSpec((tm, f_in), lambda i, j: (i, 0)),
            pl.BlockSpec((f_in, f_out), lambda i, j: (0, 0)),
            pl.BlockSpec((1, f_out), lambda i, j: (0, 0)),
        ],
        out_specs=pl.BlockSpec((tm, fo), lambda i, j: (i, j)),
        compiler_params=pltpu.CompilerParams(
            dimension_semantics=("parallel", "arbitrary"),
            vmem_limit_bytes=56 * _MB,
        ),
        cost_estimate=cost,
    )(x2, w, b2)


def kernel(x, w, b):
    f_out = w.shape[1]
    b2 = b.reshape(1, f_out)
    if x.ndim <= 2:
        x2 = x.reshape(1, -1) if x.ndim == 1 else x
        y = _linear2d(x2, w, b2)
        return y.reshape(-1) if x.ndim == 1 else y
    x2 = x.reshape(-1, x.shape[-1])
    y = _linear2d(x2, w, b2)
    return y.reshape(-1, x.shape[1], f_out)
